# baseline (speedup 1.0000x reference)
import functools

import jax
import jax.numpy as jnp
from jax import lax
from jax.experimental import pallas as pl
from jax.experimental.pallas import tpu as pltpu
from jax.experimental.pallas import tpu_sc as plsc

_CAPACITY = 100000
_KEY_SIZE = 128
_K = 50
_DELTA = 0.001

_ROWS = 10
_BLK = _CAPACITY // _ROWS

_NC = 2
_NS = 16
_NW = _NC * _NS
_PAD = 100352
_PER_W = _PAD // _NW
_VPW = _PER_W // 16
_BIG = 3.0e38


def _rowsum_mxu(xs):
    ones = jnp.ones((1, _KEY_SIZE), jnp.bfloat16)
    dims = (((1,), (1,)), ((), ()))

    def dot1(a):
        return lax.dot_general(ones, a, dims,
                               preferred_element_type=jnp.float32)

    a0 = xs.astype(jnp.bfloat16)
    r0 = xs - a0.astype(jnp.float32)
    a1 = r0.astype(jnp.bfloat16)
    r1 = r0 - a1.astype(jnp.float32)
    a2 = r1.astype(jnp.bfloat16)
    return dot1(a0) + dot1(a1) + dot1(a2)


def _tc_body(h_ref, keys_ref, dout_ref, meta_ref, dscr):
    i = pl.program_id(0)
    x = keys_ref[...] - h_ref[...]
    s2 = jnp.maximum(_rowsum_mxu(x * x), 0.0)
    drow = jnp.sqrt(s2)
    dscr[pl.ds(i, 1), :] = drow
    dout_ref[...] = drow[None]

    @pl.when(i == _ROWS - 1)
    def _():
        d = dscr[...]
        db = lax.bitcast_convert_type(d, jnp.int32)

        def _search4(data, mask, target, lo0, hi0, steps):
            def step(_, c):
                lo, hi = c
                span = hi - lo
                q = jnp.maximum(lax.div(span, jnp.int32(4)), jnp.int32(1))
                m1 = lo + q
                m2 = m1 + q
                m3 = m2 + q
                if mask is None:
                    c1 = jnp.sum((data <= m1).astype(jnp.int32))
                    c2 = jnp.sum((data <= m2).astype(jnp.int32))
                    c3 = jnp.sum((data <= m3).astype(jnp.int32))
                else:
                    c1 = jnp.sum((mask & (data <= m1)).astype(jnp.int32))
                    c2 = jnp.sum((mask & (data <= m2)).astype(jnp.int32))
                    c3 = jnp.sum((mask & (data <= m3)).astype(jnp.int32))
                lo2 = jnp.where(c3 < target, m3,
                                jnp.where(c2 < target, m2,
                                          jnp.where(c1 < target, m1, lo)))
                hi2 = jnp.where(c1 >= target, m1,
                                jnp.where(c2 >= target, m2,
                                          jnp.where(c3 >= target, m3, hi)))
                return lo2, hi2

            return lax.fori_loop(0, steps, step, (jnp.int32(lo0),
                                                  jnp.int32(hi0)))[1]

        t_bits = _search4(db, None, _K, -1, 0x7F800000, 18)
        t = lax.bitcast_convert_type(t_bits, jnp.float32)

        mask_eq = d == t
        need = _K - jnp.sum((d < t).astype(jnp.int32))
        idx = (lax.broadcasted_iota(jnp.int32, (_ROWS, _BLK), 0) * _BLK
               + lax.broadcasted_iota(jnp.int32, (_ROWS, _BLK), 1))

        p = _search4(idx, mask_eq, need, -1, 2**17 - 1, 11)

        row = lax.broadcasted_iota(jnp.int32, (2, _KEY_SIZE), 0)
        meta_ref[...] = jnp.where(row == 0, t, p.astype(jnp.float32))


@functools.partial(
    pl.kernel,
    out_type=jax.ShapeDtypeStruct((2 * _NW, 16), jnp.float32),
    mesh=plsc.VectorSubcoreMesh(core_axis_name="c", subcore_axis_name="s"),
    scratch_types=[
        pltpu.VMEM((_PER_W,), jnp.float32),
        pltpu.VMEM((_PER_W,), jnp.float32),
        pltpu.VMEM((16,), jnp.float32),
        pltpu.VMEM((16,), jnp.float32),
        pltpu.VMEM((16,), jnp.float32),
        pltpu.VMEM((16,), jnp.float32),
    ],
)
def _sc_weighted_sum(d_hbm, v_hbm, meta_hbm, out_hbm,
                     d_v, v_v, mt_v, mp_v, wacc, wvacc):
    wid = lax.axis_index("s") * _NC + lax.axis_index("c")
    base = wid * _PER_W
    pltpu.sync_copy(d_hbm.at[pl.ds(base, _PER_W)], d_v)
    pltpu.sync_copy(v_hbm.at[pl.ds(base, _PER_W)], v_v)
    pltpu.sync_copy(meta_hbm.at[pl.ds(0, 16)], mt_v)
    pltpu.sync_copy(meta_hbm.at[pl.ds(_KEY_SIZE, 16)], mp_v)
    t = mt_v[...]
    p = mp_v[...].astype(jnp.int32)
    wacc[...] = jnp.zeros((16,), jnp.float32)
    wvacc[...] = jnp.zeros((16,), jnp.float32)
    lane = lax.iota(jnp.int32, 16)

    @pl.loop(0, _VPW)
    def _(j):
        off = j * 16
        dv = d_v[pl.ds(off, 16)]
        vv = v_v[pl.ds(off, 16)]
        iv = lane + (base + off)
        sel = (dv < t) | ((dv == t) & (iv <= p))
        w = jnp.where(sel, 1.0 / (dv + _DELTA), 0.0)
        wacc[...] = wacc[...] + w
        wvacc[...] = wvacc[...] + w * vv

    pltpu.sync_copy(wacc, out_hbm.at[wid])
    pltpu.sync_copy(wvacc, out_hbm.at[wid + _NW])


def kernel(h, keys, values):
    d_out, meta = pl.pallas_call(
        _tc_body,
        grid=(_ROWS,),
        in_specs=[
            pl.BlockSpec((1, _KEY_SIZE), lambda i: (0, 0)),
            pl.BlockSpec((_BLK, _KEY_SIZE), lambda i: (i, 0)),
        ],
        out_specs=[
            pl.BlockSpec((1, 1, _BLK), lambda i: (i, 0, 0)),
            pl.BlockSpec((2, _KEY_SIZE), lambda i: (0, 0)),
        ],
        out_shape=[
            jax.ShapeDtypeStruct((_ROWS, 1, _BLK), jnp.float32),
            jax.ShapeDtypeStruct((2, _KEY_SIZE), jnp.float32),
        ],
        scratch_shapes=[pltpu.VMEM((_ROWS, _BLK), jnp.float32)],
    )(h[None, :], keys)

    d_pad = jnp.concatenate(
        [d_out.reshape(_CAPACITY), jnp.full((_PAD - _CAPACITY,), _BIG,
                                            jnp.float32)])
    v_pad = jnp.concatenate(
        [values, jnp.zeros((_PAD - _CAPACITY,), jnp.float32)])

    parts = _sc_weighted_sum(d_pad, v_pad, meta.reshape(2 * _KEY_SIZE))
    return jnp.sum(parts[_NW:]) / jnp.sum(parts[:_NW])


# speedup vs baseline: 1.0120x; 1.0120x over previous
import functools

import jax
import jax.numpy as jnp
from jax import lax
from jax.experimental import pallas as pl
from jax.experimental.pallas import tpu as pltpu
from jax.experimental.pallas import tpu_sc as plsc

_CAPACITY = 100000
_KEY_SIZE = 128
_K = 50
_DELTA = 0.001

_ROWS = 10
_BLK = _CAPACITY // _ROWS

_NC = 2
_NS = 16
_NW = _NC * _NS
_OWN = _CAPACITY // _NW
_PER_W = 3136
_VPW = _PER_W // 16
_UNROLL = 4


def _rowsum_mxu(xs):
    ones = jnp.ones((1, _KEY_SIZE), jnp.bfloat16)
    dims = (((1,), (1,)), ((), ()))

    def dot1(a):
        return lax.dot_general(ones, a, dims,
                               preferred_element_type=jnp.float32)

    a0 = xs.astype(jnp.bfloat16)
    r0 = xs - a0.astype(jnp.float32)
    a1 = r0.astype(jnp.bfloat16)
    r1 = r0 - a1.astype(jnp.float32)
    a2 = r1.astype(jnp.bfloat16)
    return dot1(a0) + dot1(a1) + dot1(a2)


def _tc_body(h_ref, keys_ref, dout_ref, meta_ref, dscr):
    i = pl.program_id(0)
    x = keys_ref[...] - h_ref[...]
    s2 = jnp.maximum(_rowsum_mxu(x * x), 0.0)
    drow = jnp.sqrt(s2)
    dscr[pl.ds(i, 1), :] = drow
    dout_ref[...] = drow[None]

    @pl.when(i == _ROWS - 1)
    def _():
        d = dscr[...]
        db = lax.bitcast_convert_type(d, jnp.int32)

        def _search4(data, mask, target, lo0, hi0, steps):
            def step(_, c):
                lo, hi = c
                span = hi - lo
                q = jnp.maximum(lax.div(span, jnp.int32(4)), jnp.int32(1))
                m1 = lo + q
                m2 = m1 + q
                m3 = m2 + q
                if mask is None:
                    c1 = jnp.sum((data <= m1).astype(jnp.int32))
                    c2 = jnp.sum((data <= m2).astype(jnp.int32))
                    c3 = jnp.sum((data <= m3).astype(jnp.int32))
                else:
                    c1 = jnp.sum((mask & (data <= m1)).astype(jnp.int32))
                    c2 = jnp.sum((mask & (data <= m2)).astype(jnp.int32))
                    c3 = jnp.sum((mask & (data <= m3)).astype(jnp.int32))
                lo2 = jnp.where(c3 < target, m3,
                                jnp.where(c2 < target, m2,
                                          jnp.where(c1 < target, m1, lo)))
                hi2 = jnp.where(c1 >= target, m1,
                                jnp.where(c2 >= target, m2,
                                          jnp.where(c3 >= target, m3, hi)))
                return lo2, hi2

            return lax.fori_loop(0, steps, step, (jnp.int32(lo0),
                                                  jnp.int32(hi0)))[1]

        t_bits = _search4(db, None, _K, -1, 0x7F800000, 18)
        t = lax.bitcast_convert_type(t_bits, jnp.float32)

        mask_eq = d == t
        need = _K - jnp.sum((d < t).astype(jnp.int32))
        idx = (lax.broadcasted_iota(jnp.int32, (_ROWS, _BLK), 0) * _BLK
               + lax.broadcasted_iota(jnp.int32, (_ROWS, _BLK), 1))

        p = _search4(idx, mask_eq, need, -1, 2**17 - 1, 11)

        row = lax.broadcasted_iota(jnp.int32, (2, _KEY_SIZE), 0)
        meta_ref[...] = jnp.where(row == 0, t, p.astype(jnp.float32))


@functools.partial(
    pl.kernel,
    out_type=jax.ShapeDtypeStruct((2 * _NW, 16), jnp.float32),
    mesh=plsc.VectorSubcoreMesh(core_axis_name="c", subcore_axis_name="s"),
    scratch_types=[
        pltpu.VMEM((_PER_W,), jnp.float32),
        pltpu.VMEM((_PER_W,), jnp.float32),
        pltpu.VMEM((16,), jnp.float32),
        pltpu.VMEM((16,), jnp.float32),
        pltpu.VMEM((16,), jnp.float32),
        pltpu.VMEM((16,), jnp.float32),
    ],
)
def _sc_weighted_sum(d_hbm, v_hbm, meta_hbm, out_hbm,
                     d_v, v_v, mt_v, mp_v, wacc, wvacc):
    wid = lax.axis_index("s") * _NC + lax.axis_index("c")
    own_lo = wid * _OWN
    own_hi = own_lo + _OWN
    abase = (own_lo // 16) * 16
    pltpu.sync_copy(d_hbm.at[pl.ds(abase, _PER_W)], d_v)
    pltpu.sync_copy(v_hbm.at[pl.ds(abase, _PER_W)], v_v)
    pltpu.sync_copy(meta_hbm.at[pl.ds(0, 16)], mt_v)
    pltpu.sync_copy(meta_hbm.at[pl.ds(_KEY_SIZE, 16)], mp_v)
    t = mt_v[...]
    p = mp_v[...].astype(jnp.int32)
    wacc[...] = jnp.zeros((16,), jnp.float32)
    wvacc[...] = jnp.zeros((16,), jnp.float32)
    lane = lax.iota(jnp.int32, 16)

    @pl.loop(0, _VPW, step=_UNROLL)
    def _(j):
        for u in range(_UNROLL):
            off = (j + u) * 16
            dv = d_v[pl.ds(off, 16)]
            vv = v_v[pl.ds(off, 16)]
            iv = lane + (abase + off)
            own = (iv >= own_lo) & (iv < own_hi)
            sel = own & ((dv < t) | ((dv == t) & (iv <= p)))
            w = jnp.where(sel, 1.0 / (dv + _DELTA), 0.0)
            wacc[...] = wacc[...] + w
            wvacc[...] = wvacc[...] + w * vv

    pltpu.sync_copy(wacc, out_hbm.at[wid])
    pltpu.sync_copy(wvacc, out_hbm.at[wid + _NW])


def kernel(h, keys, values):
    d_out, meta = pl.pallas_call(
        _tc_body,
        grid=(_ROWS,),
        in_specs=[
            pl.BlockSpec((1, _KEY_SIZE), lambda i: (0, 0)),
            pl.BlockSpec((_BLK, _KEY_SIZE), lambda i: (i, 0)),
        ],
        out_specs=[
            pl.BlockSpec((1, 1, _BLK), lambda i: (i, 0, 0)),
            pl.BlockSpec((2, _KEY_SIZE), lambda i: (0, 0)),
        ],
        out_shape=[
            jax.ShapeDtypeStruct((_ROWS, 1, _BLK), jnp.float32),
            jax.ShapeDtypeStruct((2, _KEY_SIZE), jnp.float32),
        ],
        scratch_shapes=[pltpu.VMEM((_ROWS, _BLK), jnp.float32)],
    )(h[None, :], keys)

    parts = _sc_weighted_sum(d_out.reshape(_CAPACITY), values,
                             meta.reshape(2 * _KEY_SIZE))
    return jnp.sum(parts[_NW:]) / jnp.sum(parts[:_NW])


# speedup vs baseline: 1.0964x; 1.0834x over previous
import functools

import jax
import jax.numpy as jnp
from jax import lax
from jax.experimental import pallas as pl
from jax.experimental.pallas import tpu as pltpu
from jax.experimental.pallas import tpu_sc as plsc

_CAPACITY = 100000
_KEY_SIZE = 128
_K = 50
_DELTA = 0.001

_ROWS = 10
_BLK = _CAPACITY // _ROWS

_NC = 2
_NS = 16
_NW = _NC * _NS
_OWN = _CAPACITY // _NW
_PER_W = 3136
_VPW = _PER_W // 16
_UNROLL = 4


def _rowsum_mxu(xs):
    ones = jnp.ones((1, _KEY_SIZE), jnp.bfloat16)
    dims = (((1,), (1,)), ((), ()))

    def dot1(a):
        return lax.dot_general(ones, a, dims,
                               preferred_element_type=jnp.float32)

    a0 = xs.astype(jnp.bfloat16)
    r0 = xs - a0.astype(jnp.float32)
    a1 = r0.astype(jnp.bfloat16)
    r1 = r0 - a1.astype(jnp.float32)
    a2 = r1.astype(jnp.bfloat16)
    return dot1(a0) + dot1(a1) + dot1(a2)


def _tc_body(h_ref, keys_ref, dout_ref, meta_ref, dscr):
    i = pl.program_id(0)
    x = keys_ref[...] - h_ref[...]
    s2 = jnp.maximum(_rowsum_mxu(x * x), 0.0)
    drow = jnp.sqrt(s2)
    dscr[pl.ds(i, 1), :] = drow
    dout_ref[...] = drow[None]

    @pl.when(i == _ROWS - 1)
    def _():
        d = dscr[...]
        db = lax.bitcast_convert_type(d, jnp.int32)

        def _search4(data, mask, target, lo0, hi0, steps):
            def step(_, c):
                lo, hi = c
                span = hi - lo
                q = jnp.maximum(lax.div(span, jnp.int32(4)), jnp.int32(1))
                m1 = lo + q
                m2 = m1 + q
                m3 = m2 + q
                if mask is None:
                    c1 = jnp.sum((data <= m1).astype(jnp.int32))
                    c2 = jnp.sum((data <= m2).astype(jnp.int32))
                    c3 = jnp.sum((data <= m3).astype(jnp.int32))
                else:
                    c1 = jnp.sum((mask & (data <= m1)).astype(jnp.int32))
                    c2 = jnp.sum((mask & (data <= m2)).astype(jnp.int32))
                    c3 = jnp.sum((mask & (data <= m3)).astype(jnp.int32))
                lo2 = jnp.where(c3 < target, m3,
                                jnp.where(c2 < target, m2,
                                          jnp.where(c1 < target, m1, lo)))
                hi2 = jnp.where(c1 >= target, m1,
                                jnp.where(c2 >= target, m2,
                                          jnp.where(c3 >= target, m3, hi)))
                return lo2, hi2

            return lax.fori_loop(0, steps, step, (jnp.int32(lo0),
                                                  jnp.int32(hi0)))[1]

        t_bits = _search4(db, None, _K, -1, 0x7F800000, 18)
        t = lax.bitcast_convert_type(t_bits, jnp.float32)

        mask_eq = d == t
        need = _K - jnp.sum((d < t).astype(jnp.int32))
        n_le = jnp.sum((d <= t).astype(jnp.int32))
        idx = (lax.broadcasted_iota(jnp.int32, (_ROWS, _BLK), 0) * _BLK
               + lax.broadcasted_iota(jnp.int32, (_ROWS, _BLK), 1))

        isteps = jnp.where(n_le == _K, 0, 11)
        p = _search4(idx, mask_eq, need, -1, 2**17 - 1, isteps)

        row = lax.broadcasted_iota(jnp.int32, (2, _KEY_SIZE), 0)
        meta_ref[...] = jnp.where(row == 0, t, p.astype(jnp.float32))


@functools.partial(
    pl.kernel,
    out_type=jax.ShapeDtypeStruct((2 * _NW, 16), jnp.float32),
    mesh=plsc.VectorSubcoreMesh(core_axis_name="c", subcore_axis_name="s"),
    scratch_types=[
        pltpu.VMEM((_PER_W,), jnp.float32),
        pltpu.VMEM((_PER_W,), jnp.float32),
        pltpu.VMEM((16,), jnp.float32),
        pltpu.VMEM((16,), jnp.float32),
        pltpu.VMEM((16,), jnp.float32),
        pltpu.VMEM((16,), jnp.float32),
    ],
)
def _sc_weighted_sum(d_hbm, v_hbm, meta_hbm, out_hbm,
                     d_v, v_v, mt_v, mp_v, wacc, wvacc):
    wid = lax.axis_index("s") * _NC + lax.axis_index("c")
    own_lo = wid * _OWN
    own_hi = own_lo + _OWN
    abase = (own_lo // 16) * 16
    pltpu.sync_copy(d_hbm.at[pl.ds(abase, _PER_W)], d_v)
    pltpu.sync_copy(v_hbm.at[pl.ds(abase, _PER_W)], v_v)
    pltpu.sync_copy(meta_hbm.at[pl.ds(0, 16)], mt_v)
    pltpu.sync_copy(meta_hbm.at[pl.ds(_KEY_SIZE, 16)], mp_v)
    t = mt_v[...]
    p = mp_v[...].astype(jnp.int32)
    wacc[...] = jnp.zeros((16,), jnp.float32)
    wvacc[...] = jnp.zeros((16,), jnp.float32)
    lane = lax.iota(jnp.int32, 16)

    @pl.loop(0, _VPW, step=_UNROLL)
    def _(j):
        for u in range(_UNROLL):
            off = (j + u) * 16
            dv = d_v[pl.ds(off, 16)]
            vv = v_v[pl.ds(off, 16)]
            iv = lane + (abase + off)
            own = (iv >= own_lo) & (iv < own_hi)
            sel = own & ((dv < t) | ((dv == t) & (iv <= p)))
            w = jnp.where(sel, 1.0 / (dv + _DELTA), 0.0)
            wacc[...] = wacc[...] + w
            wvacc[...] = wvacc[...] + w * vv

    pltpu.sync_copy(wacc, out_hbm.at[wid])
    pltpu.sync_copy(wvacc, out_hbm.at[wid + _NW])


def kernel(h, keys, values):
    d_out, meta = pl.pallas_call(
        _tc_body,
        grid=(_ROWS,),
        in_specs=[
            pl.BlockSpec((1, _KEY_SIZE), lambda i: (0, 0)),
            pl.BlockSpec((_BLK, _KEY_SIZE), lambda i: (i, 0)),
        ],
        out_specs=[
            pl.BlockSpec((1, 1, _BLK), lambda i: (i, 0, 0)),
            pl.BlockSpec((2, _KEY_SIZE), lambda i: (0, 0)),
        ],
        out_shape=[
            jax.ShapeDtypeStruct((_ROWS, 1, _BLK), jnp.float32),
            jax.ShapeDtypeStruct((2, _KEY_SIZE), jnp.float32),
        ],
        scratch_shapes=[pltpu.VMEM((_ROWS, _BLK), jnp.float32)],
    )(h[None, :], keys)

    parts = _sc_weighted_sum(d_out.reshape(_CAPACITY), values,
                             meta.reshape(2 * _KEY_SIZE))
    return jnp.sum(parts[_NW:]) / jnp.sum(parts[:_NW])


# speedup vs baseline: 1.5027x; 1.3706x over previous
import functools

import jax
import jax.numpy as jnp
from jax import lax
from jax.experimental import pallas as pl
from jax.experimental.pallas import tpu as pltpu

_CAPACITY = 100000
_KEY_SIZE = 128
_K = 50
_DELTA = 0.001

_ROWS = 10
_BLK = _CAPACITY // _ROWS


def _rowsum_mxu(xs):
    ones = jnp.ones((1, _KEY_SIZE), jnp.bfloat16)
    dims = (((1,), (1,)), ((), ()))

    def dot1(a):
        return lax.dot_general(ones, a, dims,
                               preferred_element_type=jnp.float32)

    a0 = xs.astype(jnp.bfloat16)
    r0 = xs - a0.astype(jnp.float32)
    a1 = r0.astype(jnp.bfloat16)
    r1 = r0 - a1.astype(jnp.float32)
    a2 = r1.astype(jnp.bfloat16)
    return dot1(a0) + dot1(a1) + dot1(a2)


def _dist_body(h_ref, keys_ref, dout_ref):
    x = keys_ref[...] - h_ref[...]
    s2 = jnp.maximum(_rowsum_mxu(x * x), 0.0)
    dout_ref[...] = jnp.sqrt(s2)[None]


def _sel_body(d_ref, vals_ref, out_ref):
    if True:
        d = d_ref[...]
        db = lax.bitcast_convert_type(d, jnp.int32)

        def _search4(data, mask, target, lo0, hi0, steps):
            def step(_, c):
                lo, hi = c
                span = hi - lo
                q = jnp.maximum(lax.div(span, jnp.int32(4)), jnp.int32(1))
                m1 = lo + q
                m2 = m1 + q
                m3 = m2 + q
                if mask is None:
                    c1 = jnp.sum((data <= m1).astype(jnp.int32))
                    c2 = jnp.sum((data <= m2).astype(jnp.int32))
                    c3 = jnp.sum((data <= m3).astype(jnp.int32))
                else:
                    c1 = jnp.sum((mask & (data <= m1)).astype(jnp.int32))
                    c2 = jnp.sum((mask & (data <= m2)).astype(jnp.int32))
                    c3 = jnp.sum((mask & (data <= m3)).astype(jnp.int32))
                lo2 = jnp.where(c3 < target, m3,
                                jnp.where(c2 < target, m2,
                                          jnp.where(c1 < target, m1, lo)))
                hi2 = jnp.where(c1 >= target, m1,
                                jnp.where(c2 >= target, m2,
                                          jnp.where(c3 >= target, m3, hi)))
                return lo2, hi2

            return lax.fori_loop(0, steps, step, (jnp.int32(lo0),
                                                  jnp.int32(hi0)))[1]

        t_bits = _search4(db, None, _K, -1, 0x7F800000, 18)
        t = lax.bitcast_convert_type(t_bits, jnp.float32)

        mask_lt = d < t
        n_lt = jnp.sum(mask_lt.astype(jnp.int32))
        need = _K - n_lt
        mask_eq = d == t
        idx = (lax.broadcasted_iota(jnp.int32, (_ROWS, _BLK), 0) * _BLK
               + lax.broadcasted_iota(jnp.int32, (_ROWS, _BLK), 1))

        p = _search4(idx, mask_eq, need, -1, 2**17 - 1, 11)

        sel = mask_lt | (mask_eq & (idx <= p))
        w = jnp.where(sel, 1.0 / (d + _DELTA), 0.0)
        acc_w = jnp.sum(w)
        acc_wv = jnp.sum(w * vals_ref[...])
        out_ref[...] = jnp.reshape(acc_wv / acc_w, (1, 1))


def kernel(h, keys, values):
    d = pl.pallas_call(
        _dist_body,
        grid=(_ROWS,),
        in_specs=[
            pl.BlockSpec((1, _KEY_SIZE), lambda i: (0, 0)),
            pl.BlockSpec((_BLK, _KEY_SIZE), lambda i: (i, 0)),
        ],
        out_specs=pl.BlockSpec((1, 1, _BLK), lambda i: (i, 0, 0)),
        out_shape=jax.ShapeDtypeStruct((_ROWS, 1, _BLK), jnp.float32),
        compiler_params=pltpu.CompilerParams(
            dimension_semantics=("parallel",)),
    )(h[None, :], keys)
    out = pl.pallas_call(
        _sel_body,
        in_specs=[
            pl.BlockSpec((_ROWS, _BLK), lambda: (0, 0)),
            pl.BlockSpec((_ROWS, _BLK), lambda: (0, 0)),
        ],
        out_specs=pl.BlockSpec((1, 1), lambda: (0, 0)),
        out_shape=jax.ShapeDtypeStruct((1, 1), jnp.float32),
    )(d.reshape(_ROWS, _BLK), values.reshape(_ROWS, _BLK))
    return out[0, 0]
